# Initial kernel scaffold; baseline (speedup 1.0000x reference)
import jax
import jax.numpy as jnp
from jax.experimental import pallas as pl


def kernel(logits, labels):
    raise NotImplementedError("write your pallas kernel here")


# speedup vs baseline: 2.2459x; 2.2459x over previous
import functools

import jax
import jax.numpy as jnp
from jax import lax
from jax.experimental import pallas as pl
from jax.experimental.pallas import tpu as pltpu
from jax.experimental.pallas import tpu_sc as plsc

_B = 1024
_C = 100000
_K = 10

_NC = 2
_NS = 16
_NW = _NC * _NS
_L = 16
_RW = 128
_BPW = _B // _NW


def _sc_gather_body(logits_hbm, labels_hbm, out_hbm, lab_v, row_v, rows_v,
                    val_v, sem):
    wid = lax.axis_index("s") * _NC + lax.axis_index("c")
    base = wid * _BPW
    pltpu.sync_copy(labels_hbm.at[pl.ds(base, _BPW)], lab_v)
    for g in range(_BPW // _L):
        lab = lab_v[pl.ds(g * _L, _L)]
        bidx = lax.iota(jnp.int32, _L) + (base + g * _L)
        flat = bidx * _C + lab
        row_v[pl.ds(g * _L, _L)] = lax.shift_right_logical(flat, 7)
    pltpu.async_copy(logits_hbm.at[row_v], rows_v, sem).wait()
    for g in range(_BPW // _L):
        lab = lab_v[pl.ds(g * _L, _L)]
        bidx = lax.iota(jnp.int32, _L) + (base + g * _L)
        lane = lax.bitwise_and(bidx * _C + lab, jnp.int32(_RW - 1))
        rowi = lax.iota(jnp.int32, _L) + g * _L
        val_v[pl.ds(g * _L, _L)] = plsc.load_gather(rows_v, [rowi, lane])
    pltpu.sync_copy(val_v, out_hbm.at[pl.ds(base, _BPW)])


def _sc_gather(logits, labels):
    mesh = plsc.VectorSubcoreMesh(core_axis_name="c", subcore_axis_name="s")
    k = functools.partial(
        pl.kernel,
        mesh=mesh,
        compiler_params=pltpu.CompilerParams(needs_layout_passes=False),
        out_type=jax.ShapeDtypeStruct((_B,), jnp.float32),
        scratch_types=[
            pltpu.VMEM((_BPW,), jnp.int32),
            pltpu.VMEM((_BPW,), jnp.int32),
            pltpu.VMEM((_BPW, _RW), jnp.float32),
            pltpu.VMEM((_BPW,), jnp.float32),
            pltpu.SemaphoreType.DMA,
        ],
    )(_sc_gather_body)
    return k(logits.reshape(_B * _C // _RW, _RW), labels)


_CHUNK = 512
_NCH = -(-_C // _CHUNK)


def _count_body(v_ref, lab_ref, x_ref, out_ref, acc_ref):
    c = pl.program_id(0)

    @pl.when(c == 0)
    def _init():
        acc_ref[...] = jnp.zeros_like(acc_ref)

    x = x_ref[...]
    v = v_ref[...]
    lab = lab_ref[...]
    col = lax.broadcasted_iota(jnp.int32, (_B, _CHUNK), 1) + c * _CHUNK
    valid = col < _C
    ahead = (valid & (x > v)) | ((x == v) & (col < lab))
    acc_ref[...] += jnp.sum(ahead.astype(jnp.float32), axis=1, keepdims=True)

    @pl.when(c == _NCH - 1)
    def _fini():
        correct = (acc_ref[...] < float(_K)).astype(jnp.float32)
        total = jnp.sum(correct) * (1.0 / _B)
        out_ref[...] = jnp.broadcast_to(total, (1, 1))


def _tc_count(logits, v, labels):
    return pl.pallas_call(
        _count_body,
        grid=(_NCH,),
        in_specs=[
            pl.BlockSpec((_B, 1), lambda c: (0, 0)),
            pl.BlockSpec((_B, 1), lambda c: (0, 0)),
            pl.BlockSpec((_B, _CHUNK), lambda c: (0, c)),
        ],
        out_specs=pl.BlockSpec((1, 1), lambda c: (0, 0)),
        out_shape=jax.ShapeDtypeStruct((1, 1), jnp.float32),
        scratch_shapes=[pltpu.VMEM((_B, 1), jnp.float32)],
    )(v.reshape(_B, 1), labels.reshape(_B, 1), logits)


def kernel(logits, labels):
    labels = labels.astype(jnp.int32)
    v = _sc_gather(logits, labels)
    out = _tc_count(logits, v, labels)
    return out.reshape(())


# speedup vs baseline: 2.5561x; 1.1381x over previous
import functools

import jax
import jax.numpy as jnp
from jax import lax
from jax.experimental import pallas as pl
from jax.experimental.pallas import tpu as pltpu
from jax.experimental.pallas import tpu_sc as plsc

_B = 1024
_C = 100000
_K = 10

_NC = 2
_NS = 16
_NW = _NC * _NS
_L = 16
_RW = 128
_BPW = _B // _NW


def _sc_gather_body(logits_hbm, labels_hbm, out_hbm, lab_v, row_v, rows_v,
                    val_v, sem):
    wid = lax.axis_index("s") * _NC + lax.axis_index("c")
    base = wid * _BPW
    pltpu.sync_copy(labels_hbm.at[pl.ds(base, _BPW)], lab_v)
    for g in range(_BPW // _L):
        lab = lab_v[pl.ds(g * _L, _L)]
        bidx = lax.iota(jnp.int32, _L) + (base + g * _L)
        flat = bidx * _C + lab
        row_v[pl.ds(g * _L, _L)] = lax.shift_right_logical(flat, 7)
    pltpu.async_copy(logits_hbm.at[row_v], rows_v, sem).wait()
    for g in range(_BPW // _L):
        lab = lab_v[pl.ds(g * _L, _L)]
        bidx = lax.iota(jnp.int32, _L) + (base + g * _L)
        lane = lax.bitwise_and(bidx * _C + lab, jnp.int32(_RW - 1))
        rowi = lax.iota(jnp.int32, _L) + g * _L
        val_v[pl.ds(g * _L, _L)] = plsc.load_gather(rows_v, [rowi, lane])
    pltpu.sync_copy(val_v, out_hbm.at[pl.ds(base, _BPW)])


def _sc_gather(logits, labels):
    mesh = plsc.VectorSubcoreMesh(core_axis_name="c", subcore_axis_name="s")
    k = functools.partial(
        pl.kernel,
        mesh=mesh,
        compiler_params=pltpu.CompilerParams(needs_layout_passes=False),
        out_type=jax.ShapeDtypeStruct((_B,), jnp.float32),
        scratch_types=[
            pltpu.VMEM((_BPW,), jnp.int32),
            pltpu.VMEM((_BPW,), jnp.int32),
            pltpu.VMEM((_BPW, _RW), jnp.float32),
            pltpu.VMEM((_BPW,), jnp.float32),
            pltpu.SemaphoreType.DMA,
        ],
    )(_sc_gather_body)
    return k(logits.reshape(_B * _C // _RW, _RW), labels)


_LANES = 128
_CHUNK = 4096
_NCH = -(-_C // _CHUNK)
_TAIL = _C - (_NCH - 1) * _CHUNK
_TAIL_FULL = _TAIL // _LANES
_TAIL_REM = _TAIL - _TAIL_FULL * _LANES

_ONE = 1.0
_ZERO = 0.0


def _count_body(v_ref, labm_ref, x_ref, out_ref, acc_ref):
    c = pl.program_id(0)

    @pl.when(c == 0)
    def _init():
        acc_ref[...] = jnp.zeros_like(acc_ref)

    vt = v_ref[...]
    labm = labm_ref[...]

    def slice_update(s, extra_mask=None):
        xs = x_ref[:, s * _LANES:(s + 1) * _LANES]
        base = c * _CHUNK + s * _LANES
        m_ge = xs >= vt
        m_gt = xs > vt
        if extra_mask is not None:
            m_ge = m_ge & extra_mask
            m_gt = m_gt & extra_mask
        mc = labm > base
        f_ge = jnp.where(m_ge, _ONE, _ZERO)
        f_gt = jnp.where(m_gt, _ONE, _ZERO)
        acc_ref[...] += jnp.where(mc, f_ge, f_gt)

    @pl.when(c < _NCH - 1)
    def _full():
        for s in range(_CHUNK // _LANES):
            slice_update(s)

    @pl.when(c == _NCH - 1)
    def _tail_and_fini():
        for s in range(_TAIL_FULL):
            slice_update(s)
        rem_mask = lax.broadcasted_iota(jnp.int32, (_B, _LANES), 1) < _TAIL_REM
        slice_update(_TAIL_FULL, extra_mask=rem_mask)
        counts = jnp.sum(acc_ref[...], axis=1, keepdims=True)
        correct = (counts < float(_K)).astype(jnp.float32)
        total = jnp.sum(correct) * (1.0 / _B)
        out_ref[...] = jnp.broadcast_to(total, (1, 1))


def _tc_count(logits, v, labels):
    vb = jnp.broadcast_to(v.reshape(_B, 1), (_B, _LANES))
    labm = labels.reshape(_B, 1) - lax.broadcasted_iota(jnp.int32, (_B, _LANES), 1)
    return pl.pallas_call(
        _count_body,
        grid=(_NCH,),
        in_specs=[
            pl.BlockSpec((_B, _LANES), lambda c: (0, 0)),
            pl.BlockSpec((_B, _LANES), lambda c: (0, 0)),
            pl.BlockSpec((_B, _CHUNK), lambda c: (0, c)),
        ],
        out_specs=pl.BlockSpec((1, 1), lambda c: (0, 0)),
        out_shape=jax.ShapeDtypeStruct((1, 1), jnp.float32),
        scratch_shapes=[pltpu.VMEM((_B, _LANES), jnp.float32)],
    )(vb, labm, logits)


def kernel(logits, labels):
    labels = labels.astype(jnp.int32)
    v = _sc_gather(logits, labels)
    out = _tc_count(logits, v, labels)
    return out.reshape(())
